# baseline (speedup 1.0000x reference)
import functools

import jax
import jax.numpy as jnp
from jax import lax
from jax.experimental import pallas as pl
from jax.experimental.pallas import tpu as pltpu
from jax.experimental.pallas import tpu_sc as plsc

EMB_D = 256
N_CODES = 8192
N_TOK = 8192
COMMIT = 0.25

TOK_TILE = 256
CODE_CHUNK = 512
N_TOK_TILES = N_TOK // TOK_TILE
N_CODE_CHUNKS = N_CODES // CODE_CHUNK

SC_NC = 2
SC_NS = 16
SC_NW = SC_NC * SC_NS
GATHER_CHUNK = 128


def _vq_body(x_ref, ew_ref, idx_ref, enc_ref, loss_ref, ent_ref, cnt_ref,
             esq_ref):
    i = pl.program_id(0)

    @pl.when(i == 0)
    def _():
        for j in range(N_CODE_CHUNKS):
            e = ew_ref[pl.ds(j * CODE_CHUNK, CODE_CHUNK), :]
            esq_ref[pl.ds(j * CODE_CHUNK, CODE_CHUNK)] = jnp.sum(e * e, axis=1)

    x = x_ref[...]
    xsq = jnp.sum(x * x, axis=1)
    run2d = jnp.full((TOK_TILE, CODE_CHUNK), jnp.inf, jnp.float32)
    chk2d = jnp.zeros((TOK_TILE, CODE_CHUNK), jnp.int32)
    for j in range(N_CODE_CHUNKS):
        e = ew_ref[pl.ds(j * CODE_CHUNK, CODE_CHUNK), :]
        esq = esq_ref[pl.ds(j * CODE_CHUNK, CODE_CHUNK)]
        m = lax.dot_general(x, e, (((1,), (1,)), ((), ())),
                            preferred_element_type=jnp.float32)
        dist = (xsq[:, None] + esq[None, :]) - 2.0 * m
        better = dist < run2d
        run2d = jnp.where(better, dist, run2d)
        chk2d = jnp.where(better, j, chk2d)
    run_min = jnp.min(run2d, axis=1)
    lane = lax.broadcasted_iota(jnp.int32, (TOK_TILE, CODE_CHUNK), 1)
    code2d = chk2d * CODE_CHUNK + lane
    cand = jnp.where(run2d == run_min[:, None], code2d, N_CODES)
    run_arg = jnp.min(cand, axis=1)
    idx_ref[0, 0, :] = run_arg

    for j in range(N_CODE_CHUNKS):
        iota = lax.broadcasted_iota(jnp.int32, (TOK_TILE, CODE_CHUNK), 1)
        enc = (iota + j * CODE_CHUNK == run_arg[:, None]).astype(jnp.float32)
        enc_ref[:, pl.ds(j * CODE_CHUNK, CODE_CHUNK)] = enc
        colsum = jnp.sum(enc, axis=0)
        sl = pl.ds(j * CODE_CHUNK, CODE_CHUNK)

        @pl.when(i == 0)
        def _():
            cnt_ref[sl] = colsum

        @pl.when(i > 0)
        def _():
            cnt_ref[sl] = cnt_ref[sl] + colsum

    partial = jnp.reshape(jnp.sum(run_min), (1, 1))

    @pl.when(i == 0)
    def _():
        loss_ref[...] = jnp.zeros((1, 1), jnp.float32)
        ent_ref[...] = jnp.zeros((1, 1), jnp.float32)

    loss_ref[...] += partial

    @pl.when(i == N_TOK_TILES - 1)
    def _():
        loss_ref[...] = loss_ref[...] * (COMMIT / (N_TOK * EMB_D))
        p = cnt_ref[...] * (1.0 / N_TOK)
        ent_ref[...] = jnp.reshape(-jnp.sum(p * jnp.log(p + 1e-10)), (1, 1))


@functools.lru_cache(maxsize=1)
def _make_sc_gather():
    mesh = plsc.VectorSubcoreMesh(core_axis_name="c", subcore_axis_name="s")

    n_chunks = (N_TOK // SC_NW) // GATHER_CHUNK

    @functools.partial(
        pl.kernel,
        mesh=mesh,
        out_type=jax.ShapeDtypeStruct((N_TOK, EMB_D), jnp.float32),
        scratch_types=[
            [pltpu.VMEM((GATHER_CHUNK,), jnp.int32) for _ in range(n_chunks)],
            [pltpu.VMEM((GATHER_CHUNK, EMB_D), jnp.float32)
             for _ in range(n_chunks)],
            pltpu.SemaphoreType.DMA,
        ],
    )
    def _sc_gather(table_hbm, idx_hbm, out_hbm, idx_vs, rows_vs, sem):
        wid = lax.axis_index("s") * SC_NC + lax.axis_index("c")
        b_per_w = N_TOK // SC_NW
        base = wid * b_per_w
        for j in range(n_chunks):
            pltpu.sync_copy(idx_hbm.at[pl.ds(base + j * GATHER_CHUNK,
                                             GATHER_CHUNK)], idx_vs[j])
        copies = [pltpu.async_copy(table_hbm.at[idx_vs[j]], rows_vs[j], sem)
                  for j in range(n_chunks)]
        for j in range(n_chunks):
            copies[j].wait()
            pltpu.sync_copy(rows_vs[j],
                            out_hbm.at[pl.ds(base + j * GATHER_CHUNK,
                                             GATHER_CHUNK)])

    return _sc_gather


def kernel(inputs, embedding_weight):
    x = jnp.transpose(inputs, (0, 2, 3, 1)).reshape(N_TOK, EMB_D)

    idx3, enc, loss11, ent11 = pl.pallas_call(
        _vq_body,
        grid=(N_TOK_TILES,),
        in_specs=[
            pl.BlockSpec((TOK_TILE, EMB_D), lambda i: (i, 0)),
            pl.BlockSpec((N_CODES, EMB_D), lambda i: (0, 0)),
        ],
        out_specs=[
            pl.BlockSpec((1, 1, TOK_TILE), lambda i: (i, 0, 0)),
            pl.BlockSpec((TOK_TILE, N_CODES), lambda i: (i, 0)),
            pl.BlockSpec((1, 1), lambda i: (0, 0)),
            pl.BlockSpec((1, 1), lambda i: (0, 0)),
        ],
        out_shape=[
            jax.ShapeDtypeStruct((N_TOK_TILES, 1, TOK_TILE), jnp.int32),
            jax.ShapeDtypeStruct((N_TOK, N_CODES), jnp.float32),
            jax.ShapeDtypeStruct((1, 1), jnp.float32),
            jax.ShapeDtypeStruct((1, 1), jnp.float32),
        ],
        scratch_shapes=[pltpu.VMEM((N_CODES,), jnp.float32),
                        pltpu.VMEM((N_CODES,), jnp.float32)],
    )(x, embedding_weight)

    q_flat = _make_sc_gather()(embedding_weight, idx3.reshape(N_TOK))
    quantized = jnp.transpose(q_flat.reshape(8, 32, 32, EMB_D), (0, 3, 1, 2))
    return (quantized, loss11[0, 0], ent11[0, 0], enc)


# speedup vs baseline: 1.0844x; 1.0844x over previous
import functools

import jax
import jax.numpy as jnp
from jax import lax
from jax.experimental import pallas as pl
from jax.experimental.pallas import tpu as pltpu
from jax.experimental.pallas import tpu_sc as plsc

EMB_D = 256
N_CODES = 8192
N_TOK = 8192
COMMIT = 0.25

TOK_TILE = 256
CODE_CHUNK = 512
N_TOK_TILES = N_TOK // TOK_TILE
N_CODE_CHUNKS = N_CODES // CODE_CHUNK

SC_NC = 2
SC_NS = 16
SC_NW = SC_NC * SC_NS
GATHER_CHUNK = 128


def _vq_body(x_ref, ew_ref, idx_ref, enc_ref, loss_ref, ent_ref, cnt_ref,
             esq_ref):
    i = pl.program_id(0)

    @pl.when(i == 0)
    def _():
        for j in range(N_CODE_CHUNKS):
            e = ew_ref[pl.ds(j * CODE_CHUNK, CODE_CHUNK), :]
            esq_ref[pl.ds(j * CODE_CHUNK, CODE_CHUNK)] = jnp.sum(e * e, axis=1)

    x = x_ref[0]
    xsq = jnp.sum(x * x, axis=0)
    run2d = jnp.full((CODE_CHUNK, TOK_TILE), jnp.inf, jnp.float32)
    chk2d = jnp.zeros((CODE_CHUNK, TOK_TILE), jnp.int32)
    for j in range(N_CODE_CHUNKS):
        e = ew_ref[pl.ds(j * CODE_CHUNK, CODE_CHUNK), :]
        esq = esq_ref[pl.ds(j * CODE_CHUNK, CODE_CHUNK)]
        m = lax.dot_general(e, x, (((1,), (0,)), ((), ())),
                            preferred_element_type=jnp.float32)
        dist = (xsq[None, :] + esq[:, None]) - 2.0 * m
        better = dist < run2d
        run2d = jnp.where(better, dist, run2d)
        chk2d = jnp.where(better, j, chk2d)
    run_min = jnp.min(run2d, axis=0)
    slot = lax.broadcasted_iota(jnp.int32, (CODE_CHUNK, TOK_TILE), 0)
    code2d = chk2d * CODE_CHUNK + slot
    cand = jnp.where(run2d == run_min[None, :], code2d, N_CODES)
    run_arg = jnp.min(cand, axis=0)
    idx_ref[0, 0, :] = run_arg

    arg_col = jnp.transpose(jnp.reshape(run_arg, (1, TOK_TILE)))
    for j in range(N_CODE_CHUNKS):
        iota = lax.broadcasted_iota(jnp.int32, (TOK_TILE, CODE_CHUNK), 1)
        enc = (iota + j * CODE_CHUNK == arg_col).astype(jnp.float32)
        enc_ref[:, pl.ds(j * CODE_CHUNK, CODE_CHUNK)] = enc
        colsum = jnp.sum(enc, axis=0)
        sl = pl.ds(j * CODE_CHUNK, CODE_CHUNK)

        @pl.when(i == 0)
        def _():
            cnt_ref[sl] = colsum

        @pl.when(i > 0)
        def _():
            cnt_ref[sl] = cnt_ref[sl] + colsum

    partial = jnp.reshape(jnp.sum(run_min), (1, 1))

    @pl.when(i == 0)
    def _():
        loss_ref[...] = jnp.zeros((1, 1), jnp.float32)
        ent_ref[...] = jnp.zeros((1, 1), jnp.float32)

    loss_ref[...] += partial

    @pl.when(i == N_TOK_TILES - 1)
    def _():
        loss_ref[...] = loss_ref[...] * (COMMIT / (N_TOK * EMB_D))
        p = cnt_ref[...] * (1.0 / N_TOK)
        ent_ref[...] = jnp.reshape(-jnp.sum(p * jnp.log(p + 1e-10)), (1, 1))


@functools.lru_cache(maxsize=1)
def _make_sc_gather():
    mesh = plsc.VectorSubcoreMesh(core_axis_name="c", subcore_axis_name="s")

    n_chunks = (N_TOK // SC_NW) // GATHER_CHUNK

    @functools.partial(
        pl.kernel,
        mesh=mesh,
        out_type=jax.ShapeDtypeStruct((N_TOK, EMB_D), jnp.float32),
        scratch_types=[
            pltpu.VMEM((GATHER_CHUNK,), jnp.int32),
            pltpu.VMEM((GATHER_CHUNK, EMB_D), jnp.float32),
            pltpu.SemaphoreType.DMA,
        ],
    )
    def _sc_gather(table_hbm, idx_hbm, out_hbm, idx_v, rows_v, sem):
        wid = lax.axis_index("s") * SC_NC + lax.axis_index("c")
        b_per_w = N_TOK // SC_NW
        base = wid * b_per_w
        for j in range(n_chunks):
            off = base + j * GATHER_CHUNK
            pltpu.sync_copy(idx_hbm.at[pl.ds(off, GATHER_CHUNK)], idx_v)
            pltpu.async_copy(table_hbm.at[idx_v], rows_v, sem).wait()
            pltpu.sync_copy(rows_v, out_hbm.at[pl.ds(off, GATHER_CHUNK)])

    return _sc_gather


def kernel(inputs, embedding_weight):
    x3 = inputs.reshape(8, EMB_D, 1024)

    idx3, enc, loss11, ent11 = pl.pallas_call(
        _vq_body,
        grid=(N_TOK_TILES,),
        in_specs=[
            pl.BlockSpec((1, EMB_D, TOK_TILE),
                         lambda i: (i // (1024 // TOK_TILE), 0,
                                    i % (1024 // TOK_TILE))),
            pl.BlockSpec((N_CODES, EMB_D), lambda i: (0, 0)),
        ],
        out_specs=[
            pl.BlockSpec((1, 1, TOK_TILE), lambda i: (i, 0, 0)),
            pl.BlockSpec((TOK_TILE, N_CODES), lambda i: (i, 0)),
            pl.BlockSpec((1, 1), lambda i: (0, 0)),
            pl.BlockSpec((1, 1), lambda i: (0, 0)),
        ],
        out_shape=[
            jax.ShapeDtypeStruct((N_TOK_TILES, 1, TOK_TILE), jnp.int32),
            jax.ShapeDtypeStruct((N_TOK, N_CODES), jnp.float32),
            jax.ShapeDtypeStruct((1, 1), jnp.float32),
            jax.ShapeDtypeStruct((1, 1), jnp.float32),
        ],
        scratch_shapes=[pltpu.VMEM((N_CODES,), jnp.float32),
                        pltpu.VMEM((N_CODES,), jnp.float32)],
    )(x3, embedding_weight)

    q_flat = _make_sc_gather()(embedding_weight, idx3.reshape(N_TOK))
    quantized = jnp.transpose(q_flat.reshape(8, 32, 32, EMB_D), (0, 3, 1, 2))
    return (quantized, loss11[0, 0], ent11[0, 0], enc)
